# baseline (speedup 1.0000x reference)
import functools

import jax
import jax.numpy as jnp
from jax import lax
from jax.experimental import pallas as pl
from jax.experimental.pallas import tpu as pltpu
from jax.experimental.pallas import tpu_sc as plsc

B = 16384
L = 200
E = 64
NW = 32
XT = B * L
CH = 128
NCH = XT // CH
CH_PER_W = NCH // NW
LANES = 16

_mesh = plsc.VectorSubcoreMesh(core_axis_name="c", subcore_axis_name="s")


@functools.partial(
    pl.kernel,
    mesh=_mesh,
    out_type=jax.ShapeDtypeStruct((XT, E), jnp.float32),
    scratch_types=[
        pltpu.VMEM((2 * L, E), jnp.float32),
        pltpu.VMEM((CH,), jnp.int32),
        pltpu.VMEM((CH, E), jnp.float32),
        pltpu.SemaphoreType.DMA,
    ],
    compiler_params=pltpu.CompilerParams(use_tc_tiling_on_sc=False),
)
def _embed(x_hbm, emb_hbm, pos_hbm, out_hbm, pos_v, idx_v, rows_v, sem):
    wid = lax.axis_index("s") * 2 + lax.axis_index("c")
    pltpu.sync_copy(pos_hbm, pos_v)
    base = wid * CH_PER_W

    def body_chunk(t, carry):
        s = (base + t) * CH
        p0 = lax.rem(s, L)
        pltpu.sync_copy(x_hbm.at[pl.ds(s, CH)], idx_v)
        pltpu.async_copy(emb_hbm.at[idx_v], rows_v, sem).wait()

        def body_r(r, carry3):
            for k in range(E // LANES):
                sl = pl.ds(k * LANES, LANES)
                plsc.addupdate(rows_v.at[r, sl], pos_v[p0 + r, sl])
            return carry3

        lax.fori_loop(0, CH, body_r, 0, unroll=2)
        pltpu.sync_copy(rows_v, out_hbm.at[pl.ds(s, CH)])
        return carry

    lax.fori_loop(0, CH_PER_W, body_chunk, 0)


def kernel(x, embedding, position_embedding):
    x_flat = x.reshape(XT)
    pos_ext = jnp.concatenate([position_embedding, position_embedding], axis=0)
    out = _embed(x_flat, embedding, pos_ext)
    return out.reshape(B, L, E)


# speedup vs baseline: 1.7768x; 1.7768x over previous
import functools

import jax
import jax.numpy as jnp
from jax import lax
from jax.experimental import pallas as pl
from jax.experimental.pallas import tpu as pltpu
from jax.experimental.pallas import tpu_sc as plsc

B = 16384
L = 200
E = 64
NW = 32
XT = B * L
MC = 200
G = 2
GC = MC // G
NMAC = XT // MC
M_PER_W = NMAC // NW
NBUF = 4
LANES = 16

_mesh = plsc.VectorSubcoreMesh(core_axis_name="c", subcore_axis_name="s")


@functools.partial(
    pl.kernel,
    mesh=_mesh,
    out_type=jax.ShapeDtypeStruct((XT, E), jnp.float32),
    scratch_types=[
        pltpu.VMEM((L, E), jnp.float32),
        [pltpu.VMEM((G, GC), jnp.int32) for _ in range(NBUF)],
        [pltpu.VMEM((MC, E), jnp.float32) for _ in range(NBUF)],
        [pltpu.SemaphoreType.DMA for _ in range(NBUF)],
        [pltpu.SemaphoreType.DMA for _ in range(NBUF)],
        [pltpu.SemaphoreType.DMA for _ in range(NBUF)],
    ],
    compiler_params=pltpu.CompilerParams(use_tc_tiling_on_sc=False),
)
def _embed(x_hbm, emb_hbm, pos_hbm, out_hbm, pos_v, idxs, rows, isems, gsems, osems):
    wid = lax.axis_index("s") * 2 + lax.axis_index("c")
    pltpu.sync_copy(pos_hbm, pos_v)
    base = wid * M_PER_W

    def idx_start(m, slot):
        pltpu.async_copy(x_hbm.at[base + m], idxs[slot], isems[slot])

    def idx_wait(slot):
        pltpu.make_async_copy(x_hbm.at[0], idxs[slot], isems[slot]).wait()

    def gather_start(slot):
        for g in range(G):
            pltpu.async_copy(
                emb_hbm.at[idxs[slot].at[g]],
                rows[slot].at[pl.ds(g * GC, GC)],
                gsems[slot],
            )

    def gather_wait(slot):
        pltpu.make_async_copy(
            emb_hbm.at[pl.ds(0, MC)], rows[slot], gsems[slot]
        ).wait()

    def out_start(m, slot):
        pltpu.async_copy(
            rows[slot], out_hbm.at[pl.ds((base + m) * MC, MC)], osems[slot]
        )

    def out_wait(slot):
        pltpu.make_async_copy(
            rows[slot], out_hbm.at[pl.ds(0, MC)], osems[slot]
        ).wait()

    def add_pos(slot):
        def add_body(r, c):
            for k in range(E // LANES):
                sl = pl.ds(k * LANES, LANES)
                plsc.addupdate(rows[slot].at[r, sl], pos_v[r, sl])
            return c

        lax.fori_loop(0, MC, add_body, 0, unroll=4)

    idx_start(0, 0)
    idx_start(1, 1)
    idx_wait(0)
    gather_start(0)

    def step(m, slot_m):
        slot1 = (slot_m + 1) % NBUF
        slot2 = (slot_m + 2) % NBUF

        @pl.when(m + 2 < M_PER_W)
        def _():
            idx_start(m + 2, slot2)

        @pl.when(m + 1 < M_PER_W)
        def _():
            idx_wait(slot1)

            @pl.when(m + 1 >= NBUF)
            def _():
                out_wait(slot1)

            gather_start(slot1)

        gather_wait(slot_m)
        add_pos(slot_m)
        out_start(m, slot_m)

    def body(i, carry):
        for k in range(NBUF):
            step(i * NBUF + k, k)
        return carry

    lax.fori_loop(0, M_PER_W // NBUF, body, 0)

    for k in range(NBUF - 1):
        out_wait((M_PER_W - 1 - k) % NBUF)


def kernel(x, embedding, position_embedding):
    x3 = x.reshape(NMAC, G, GC)
    out = _embed(x3, embedding, position_embedding)
    return out.reshape(B, L, E)


# speedup vs baseline: 1.8100x; 1.0187x over previous
import functools

import jax
import jax.numpy as jnp
from jax import lax
from jax.experimental import pallas as pl
from jax.experimental.pallas import tpu as pltpu
from jax.experimental.pallas import tpu_sc as plsc

B = 16384
L = 200
E = 64
NW = 32
XT = B * L
MC = 200
G = 2
GC = MC // G
NMAC = XT // MC
M_PER_W = NMAC // NW
NBUF = 8
BLK = 8
NBLK = M_PER_W // BLK
GLA = 3
LANES = 16

_mesh = plsc.VectorSubcoreMesh(core_axis_name="c", subcore_axis_name="s")


@functools.partial(
    pl.kernel,
    mesh=_mesh,
    out_type=jax.ShapeDtypeStruct((XT, E), jnp.float32),
    scratch_types=[
        pltpu.VMEM((L, E), jnp.float32),
        [pltpu.VMEM((BLK, G, GC), jnp.int32) for _ in range(2)],
        [pltpu.VMEM((MC, E), jnp.float32) for _ in range(NBUF)],
        [pltpu.SemaphoreType.DMA for _ in range(2)],
        [pltpu.SemaphoreType.DMA for _ in range(NBUF)],
        [pltpu.SemaphoreType.DMA for _ in range(NBUF)],
    ],
    compiler_params=pltpu.CompilerParams(use_tc_tiling_on_sc=False),
)
def _embed(x_hbm, emb_hbm, pos_hbm, out_hbm, pos_v, idxs, rows, isems, gsems, osems):
    wid = lax.axis_index("s") * 2 + lax.axis_index("c")
    pltpu.sync_copy(pos_hbm, pos_v)
    blk_base = wid * NBLK

    def idx_start(blk, slot):
        pltpu.async_copy(x_hbm.at[blk_base + blk], idxs[slot], isems[slot])

    def idx_wait(slot):
        pltpu.make_async_copy(x_hbm.at[0], idxs[slot], isems[slot]).wait()

    def gather_start(m, mslot, islot, j):
        for g in range(G):
            pltpu.async_copy(
                emb_hbm.at[idxs[islot].at[j, g]],
                rows[mslot].at[pl.ds(g * GC, GC)],
                gsems[mslot],
            )

    def gather_wait(mslot):
        pltpu.make_async_copy(
            emb_hbm.at[pl.ds(0, MC)], rows[mslot], gsems[mslot]
        ).wait()

    def out_start(m, mslot):
        pltpu.async_copy(
            rows[mslot],
            out_hbm.at[pl.ds((blk_base * BLK + m) * MC, MC)],
            osems[mslot],
        )

    def out_wait(mslot):
        pltpu.make_async_copy(
            rows[mslot], out_hbm.at[pl.ds(0, MC)], osems[mslot]
        ).wait()

    def add_pos(mslot):
        def add_body(r, c):
            for k in range(E // LANES):
                sl = pl.ds(k * LANES, LANES)
                plsc.addupdate(rows[mslot].at[r, sl], pos_v[r, sl])
            return c

        lax.fori_loop(0, MC, add_body, 0, unroll=4)

    idx_start(0, 0)
    idx_wait(0)
    for m in range(GLA):
        gather_start(m, m, 0, m)

    def step(blk, half, kk):
        m = blk * BLK + kk
        mslot = kk

        if kk == 0:
            @pl.when(blk + 1 < NBLK)
            def _():
                idx_start(blk + 1, 1 - half)

        @pl.when(m + GLA < M_PER_W)
        def _():
            if kk == BLK - GLA:
                idx_wait(1 - half)

            @pl.when(m + GLA >= NBUF)
            def _():
                out_wait((kk + GLA) % NBUF)

            gather_start(m + GLA, (kk + GLA) % NBUF,
                         (half + (kk + GLA) // BLK) % 2, (kk + GLA) % BLK)

        gather_wait(mslot)
        add_pos(mslot)
        out_start(m, mslot)

    def body(p, carry):
        for half in range(2):
            blk = p * 2 + half
            for kk in range(BLK):
                step(blk, half, kk)
        return carry

    lax.fori_loop(0, NBLK // 2, body, 0)

    for k in range(GLA, NBUF):
        out_wait(k)


def kernel(x, embedding, position_embedding):
    x4 = x.reshape(NMAC // BLK, BLK, G, GC)
    out = _embed(x4, embedding, position_embedding)
    return out.reshape(B, L, E)
